# baseline (speedup 1.0000x reference)
import jax
import jax.numpy as jnp
from jax import lax
from jax.experimental import pallas as pl
from jax.experimental.pallas import tpu as pltpu
from jax.experimental.pallas import tpu_sc as plsc

_T = 16
_BLK = 256
_NC, _NS, _L = 2, 16, 16
_NW = _NC * _NS
_ACC = 8


def _sc_minmax_body(x_ref, mins_ref, maxs_ref, xv_ref, mn_ref, mx_ref):
    c = lax.axis_index("c")
    s = lax.axis_index("s")
    wid = s * _NC + c
    n = xv_ref.shape[0]
    base = wid * n
    pltpu.sync_copy(x_ref.at[pl.ds(base, n)], xv_ref)

    init = tuple(xv_ref[pl.ds(j * _L, _L)] for j in range(2 * _ACC))

    def body(i, carry):
        out = []
        for j in range(_ACC):
            v = xv_ref[pl.ds(i * (_ACC * _L) + j * _L, _L)]
            out.append(jnp.minimum(carry[j], v))
            out.append(jnp.maximum(carry[_ACC + j], v))
        return tuple(out[::2]) + tuple(out[1::2])

    carry = lax.fori_loop(1, n // (_ACC * _L), body, init)
    mn = carry[0]
    mx = carry[_ACC]
    for j in range(1, _ACC):
        mn = jnp.minimum(mn, carry[j])
        mx = jnp.maximum(mx, carry[_ACC + j])
    mn_ref[...] = mn
    mx_ref[...] = mx
    pltpu.sync_copy(mn_ref, mins_ref.at[wid])
    pltpu.sync_copy(mx_ref, maxs_ref.at[wid])


def _encode_body(mins_ref, maxs_ref, x_ref, out_ref):
    mn = jnp.min(mins_ref[...])
    mx = jnp.max(maxs_ref[...])
    x = x_ref[...]
    xn = jnp.clip((x - mn) / (mx - mn + 1e-8), 0.0, 1.0)
    lat = ((1.0 - xn) * (_T - 1)).astype(jnp.int32)
    t = jax.lax.broadcasted_iota(jnp.int32, (x.shape[0], _T, x.shape[1]), 1)
    out_ref[...] = (lat[:, None, :] == t).astype(jnp.float32)


def kernel(x):
    B, F = x.shape
    n_per_w = (B * F) // _NW

    sc_minmax = pl.kernel(
        _sc_minmax_body,
        out_type=(
            jax.ShapeDtypeStruct((_NW, _L), jnp.float32),
            jax.ShapeDtypeStruct((_NW, _L), jnp.float32),
        ),
        mesh=plsc.VectorSubcoreMesh(
            core_axis_name="c", subcore_axis_name="s",
            num_cores=_NC, num_subcores=_NS,
        ),
        scratch_types=[
            pltpu.VMEM((n_per_w,), jnp.float32),
            pltpu.VMEM((_L,), jnp.float32),
            pltpu.VMEM((_L,), jnp.float32),
        ],
    )
    mins, maxs = sc_minmax(x.reshape(B * F))

    spikes = pl.pallas_call(
        _encode_body,
        grid=(B // _BLK,),
        in_specs=(
            pl.BlockSpec((_NW, _L), lambda i: (0, 0)),
            pl.BlockSpec((_NW, _L), lambda i: (0, 0)),
            pl.BlockSpec((_BLK, F), lambda i: (i, 0)),
        ),
        out_specs=pl.BlockSpec((_BLK, _T, F), lambda i: (i, 0, 0)),
        out_shape=jax.ShapeDtypeStruct((B, _T, F), jnp.float32),
    )(mins, maxs, x)
    return spikes


# speedup vs baseline: 1.1584x; 1.1584x over previous
import jax
import jax.numpy as jnp
from jax import lax
from jax.experimental import pallas as pl
from jax.experimental.pallas import tpu as pltpu
from jax.experimental.pallas import tpu_sc as plsc

_T = 16
_BLK = 256
_NC, _NS, _L = 2, 16, 16
_NW = _NC * _NS
_ACC = 8


def _sc_minmax_body(x_ref, mins_ref, maxs_ref, xv_ref, mn_ref, mx_ref):
    c = lax.axis_index("c")
    s = lax.axis_index("s")
    wid = s * _NC + c
    rows, f = xv_ref.shape
    base = wid * rows
    pltpu.sync_copy(x_ref.at[pl.ds(base, rows)], xv_ref)

    per_row = f // _L
    init = tuple(xv_ref[0, pl.ds(j * _L, _L)] for j in range(2 * _ACC))

    def body(i, carry):
        out = []
        r = i // (per_row // _ACC)
        k = (i % (per_row // _ACC)) * _ACC
        for j in range(_ACC):
            v = xv_ref[r, pl.ds((k + j) * _L, _L)]
            out.append(jnp.minimum(carry[j], v))
            out.append(jnp.maximum(carry[_ACC + j], v))
        return tuple(out[::2]) + tuple(out[1::2])

    carry = lax.fori_loop(1, (rows * per_row) // _ACC, body, init)
    mn = carry[0]
    mx = carry[_ACC]
    for j in range(1, _ACC):
        mn = jnp.minimum(mn, carry[j])
        mx = jnp.maximum(mx, carry[_ACC + j])
    mn_ref[...] = mn
    mx_ref[...] = mx
    pltpu.sync_copy(mn_ref, mins_ref.at[wid])
    pltpu.sync_copy(mx_ref, maxs_ref.at[wid])


def _encode_body(mins_ref, maxs_ref, x_ref, out_ref):
    mn = jnp.min(mins_ref[...])
    mx = jnp.max(maxs_ref[...])
    x = x_ref[...]
    xn = jnp.clip((x - mn) / (mx - mn + 1e-8), 0.0, 1.0)
    lat = ((1.0 - xn) * (_T - 1)).astype(jnp.int32)
    t = jax.lax.broadcasted_iota(jnp.int32, (x.shape[0], _T, x.shape[1]), 1)
    out_ref[...] = (lat[:, None, :] == t).astype(jnp.float32)


def kernel(x):
    B, F = x.shape
    rows_per_w = B // _NW

    sc_minmax = pl.kernel(
        _sc_minmax_body,
        out_type=(
            jax.ShapeDtypeStruct((_NW, _L), jnp.float32),
            jax.ShapeDtypeStruct((_NW, _L), jnp.float32),
        ),
        mesh=plsc.VectorSubcoreMesh(
            core_axis_name="c", subcore_axis_name="s",
            num_cores=_NC, num_subcores=_NS,
        ),
        scratch_types=[
            pltpu.VMEM((rows_per_w, F), jnp.float32),
            pltpu.VMEM((_L,), jnp.float32),
            pltpu.VMEM((_L,), jnp.float32),
        ],
    )
    mins, maxs = sc_minmax(x)

    spikes = pl.pallas_call(
        _encode_body,
        grid=(B // _BLK,),
        in_specs=(
            pl.BlockSpec((_NW, _L), lambda i: (0, 0)),
            pl.BlockSpec((_NW, _L), lambda i: (0, 0)),
            pl.BlockSpec((_BLK, F), lambda i: (i, 0)),
        ),
        out_specs=pl.BlockSpec((_BLK, _T, F), lambda i: (i, 0, 0)),
        out_shape=jax.ShapeDtypeStruct((B, _T, F), jnp.float32),
    )(mins, maxs, x)
    return spikes


# speedup vs baseline: 1.6958x; 1.4639x over previous
import jax
import jax.numpy as jnp
from jax.experimental import pallas as pl
from jax.experimental.pallas import tpu as pltpu

_T = 16
_BLK = 256


def _body(x_hbm, out_hbm, xv, ob0, ob1, insem, s0, s1):
    B, F = xv.shape
    nblk = B // _BLK

    cp_in = pltpu.make_async_copy(x_hbm, xv, insem)
    cp_in.start()
    cp_in.wait()

    mn = jnp.min(xv[...])
    mx = jnp.max(xv[...])
    scale = mx - mn + 1e-8

    bufs = (ob0, ob1)
    sems = (s0, s1)
    for i in range(nblk):
        buf = bufs[i % 2]
        sem = sems[i % 2]
        if i >= 2:
            pltpu.make_async_copy(
                buf, out_hbm.at[pl.ds((i - 2) * _BLK, _BLK)], sem
            ).wait()
        xblk = xv[pl.ds(i * _BLK, _BLK), :]
        xn = jnp.clip((xblk - mn) / scale, 0.0, 1.0)
        lat = ((1.0 - xn) * (_T - 1)).astype(jnp.int32)
        t = jax.lax.broadcasted_iota(jnp.int32, (_BLK, _T, F), 1)
        buf[...] = (lat[:, None, :] == t).astype(jnp.float32)
        pltpu.make_async_copy(
            buf, out_hbm.at[pl.ds(i * _BLK, _BLK)], sem
        ).start()
    for i in range(nblk - 2, nblk):
        pltpu.make_async_copy(
            bufs[i % 2], out_hbm.at[pl.ds(i * _BLK, _BLK)], sems[i % 2]
        ).wait()


def kernel(x):
    B, F = x.shape
    return pl.pallas_call(
        _body,
        in_specs=(pl.BlockSpec(memory_space=pl.ANY),),
        out_specs=pl.BlockSpec(memory_space=pl.ANY),
        out_shape=jax.ShapeDtypeStruct((B, _T, F), jnp.float32),
        scratch_shapes=[
            pltpu.VMEM((B, F), jnp.float32),
            pltpu.VMEM((_BLK, _T, F), jnp.float32),
            pltpu.VMEM((_BLK, _T, F), jnp.float32),
            pltpu.SemaphoreType.DMA,
            pltpu.SemaphoreType.DMA,
            pltpu.SemaphoreType.DMA,
        ],
    )(x)
